# Initial kernel scaffold; baseline (speedup 1.0000x reference)
import jax
import jax.numpy as jnp
from jax.experimental import pallas as pl


def kernel(x, edge_index, batch, params):
    raise NotImplementedError("write your pallas kernel here")


# speedup vs baseline: 4.5823x; 4.5823x over previous
import functools

import jax
import jax.numpy as jnp
from jax import lax
from jax.experimental import pallas as pl
from jax.experimental.pallas import tpu as pltpu
from jax.experimental.pallas import tpu_sc as plsc

_BN_EPS = 1e-5
_NC = 2
_NS = 16
_NW = _NC * _NS
_CHUNK = 128
_G = 64


def _sc_aggregate(h, src3, dst3, n_pad):
    n, hdim = h.shape
    _, n_chunks, chunk = src3.shape
    rows_per_tile = n_pad // _NS

    mesh = plsc.VectorSubcoreMesh(core_axis_name="c", subcore_axis_name="s",
                                  num_cores=_NC, num_subcores=_NS)

    @functools.partial(
        pl.kernel,
        out_type=jax.ShapeDtypeStruct((_NC, n_pad, hdim), jnp.float32),
        mesh=mesh,
        scratch_types=[
            pltpu.VMEM((n_chunks, chunk), jnp.int32),
            pltpu.VMEM((n_chunks, chunk), jnp.int32),
            pltpu.VMEM((chunk, hdim), jnp.float32),
            pltpu.VMEM_SHARED((n_pad, hdim), jnp.float32),
        ],
    )
    def k(h_hbm, src_hbm, dst_hbm, out_hbm, src_v, dst_v, rows_v, agg_sh):
        cid = lax.axis_index("c")
        sid = lax.axis_index("s")
        w = cid * _NS + sid

        pltpu.sync_copy(src_hbm.at[w], src_v)
        pltpu.sync_copy(dst_hbm.at[w], dst_v)

        @pl.loop(0, chunk)
        def _(r):
            @pl.loop(0, hdim, step=16)
            def _(c0):
                rows_v[r, pl.ds(c0, 16)] = jnp.zeros((16,), jnp.float32)

        @pl.loop(0, rows_per_tile, step=chunk)
        def _(j):
            pltpu.sync_copy(rows_v, agg_sh.at[pl.ds(sid * rows_per_tile + j, chunk)])

        plsc.subcore_barrier()

        @pl.loop(0, n_chunks)
        def _(c):
            pltpu.sync_copy(h_hbm.at[src_v.at[c]], rows_v)
            pltpu.sync_copy(rows_v, agg_sh.at[dst_v.at[c]], add=True)

        plsc.subcore_barrier()

        @pl.loop(0, rows_per_tile, step=chunk)
        def _(j):
            r = sid * rows_per_tile + j
            pltpu.sync_copy(agg_sh.at[pl.ds(r, chunk)], out_hbm.at[cid, pl.ds(r, chunk)])

    return k(h, src3, dst3)


def _mlp_body(agg_ref, h_ref, coef_ref, w1_ref, b1_ref, w2_ref, b2_ref,
              sc_ref, sh_ref, o_ref):
    a = agg_ref[0] + agg_ref[1] + coef_ref[...] * h_ref[...]
    z = jnp.dot(a, w1_ref[...], preferred_element_type=jnp.float32)
    z = jnp.maximum(z + b1_ref[...], 0.0)
    z = jnp.dot(z, w2_ref[...], preferred_element_type=jnp.float32)
    z = jnp.maximum(z + b2_ref[...], 0.0)
    o_ref[...] = z * sc_ref[...] + sh_ref[...]


def _mlp(aggp, h, coef_row, w1, b1, w2, b2, scale, shift, blk=2000):
    n, hd = h.shape
    grid = (n // blk,)
    full = lambda i: (0, 0)
    return pl.pallas_call(
        _mlp_body,
        grid=grid,
        in_specs=[
            pl.BlockSpec((_NC, blk, hd), lambda i: (0, i, 0)),
            pl.BlockSpec((blk, hd), lambda i: (i, 0)),
            pl.BlockSpec((1, hd), full),
            pl.BlockSpec((hd, hd), full),
            pl.BlockSpec((1, hd), full),
            pl.BlockSpec((hd, hd), full),
            pl.BlockSpec((1, hd), full),
            pl.BlockSpec((1, hd), full),
            pl.BlockSpec((1, hd), full),
        ],
        out_specs=pl.BlockSpec((blk, hd), lambda i: (i, 0)),
        out_shape=jax.ShapeDtypeStruct((n, hd), jnp.float32),
    )(aggp, h, coef_row, w1, b1, w2, b2, scale, shift)


def _pool_body(batch_ref, h_ref, w1_ref, b1_ref, w2_ref, b2_ref,
               out_ref, ge_ref):
    seg = lax.broadcasted_iota(jnp.int32, (1, _G), 1)
    onehot = (batch_ref[...] == seg).astype(jnp.float32)
    sums = lax.dot_general(onehot, h_ref[...],
                           (((0,), (0,)), ((), ())),
                           preferred_element_type=jnp.float32)
    counts = jnp.sum(onehot, axis=0)[:, None]
    ge = sums / jnp.maximum(counts, 1.0)
    z = jnp.dot(ge, w1_ref[...], preferred_element_type=jnp.float32)
    z = jnp.maximum(z + b1_ref[...], 0.0)
    out = jnp.dot(z, w2_ref[...], preferred_element_type=jnp.float32) + b2_ref[...]
    ge_ref[...] = ge
    out_ref[...] = out


def _pool_head(batch2, h, w1, b1, w2, b2):
    n, hd = h.shape
    c = w2.shape[1]
    return pl.pallas_call(
        _pool_body,
        out_shape=(
            jax.ShapeDtypeStruct((_G, c), jnp.float32),
            jax.ShapeDtypeStruct((_G, hd), jnp.float32),
        ),
    )(batch2, h, w1, b1, w2, b2)


def kernel(x, edge_index, batch, params):
    n, _ = x.shape
    e = edge_index.shape[1]
    hd = params["l0"]["W1"].shape[1]

    per_w = -(-e // (_NW * _CHUNK)) * _CHUNK
    n_chunks = per_w // _CHUNK
    e_pad = _NW * per_w
    n_pad = -(-n // (_NS * _CHUNK)) * (_NS * _CHUNK)

    src = edge_index[0].astype(jnp.int32)
    dst = edge_index[1].astype(jnp.int32)
    pad = e_pad - e
    src3 = jnp.concatenate([src, jnp.zeros((pad,), jnp.int32)]).reshape(
        _NW, n_chunks, _CHUNK)
    dst3 = jnp.concatenate([dst, jnp.full((pad,), n, jnp.int32)]).reshape(
        _NW, n_chunks, _CHUNK)

    h = x
    for l in range(3):
        p = params["l%d" % l]
        aggp = _sc_aggregate(h, src3, dst3, n_pad)
        coef = jnp.broadcast_to((1.0 + p["eps"]).reshape(1, 1), (1, hd))
        scale = (p["gamma"] / jnp.sqrt(p["var"] + _BN_EPS)).reshape(1, hd)
        shift = p["beta"].reshape(1, hd) - p["mean"].reshape(1, hd) * scale
        h = _mlp(aggp, h, coef,
                 p["W1"], p["b1"].reshape(1, hd),
                 p["W2"], p["b2"].reshape(1, hd),
                 scale, shift)

    out, ge = _pool_head(batch.astype(jnp.int32).reshape(n, 1), h,
                         params["lin1_W"], params["lin1_b"].reshape(1, hd),
                         params["lin2_W"], params["lin2_b"].reshape(1, -1))
    return (out, h, ge)


# speedup vs baseline: 13.1234x; 2.8639x over previous
import functools

import jax
import jax.numpy as jnp
from jax import lax
from jax.experimental import pallas as pl
from jax.experimental.pallas import tpu as pltpu
from jax.experimental.pallas import tpu_sc as plsc

_BN_EPS = 1e-5
_NC = 2
_NS = 16
_NW = _NC * _NS
_CHUNK = 128
_G = 64


def _sc_aggregate(h, src2, dst2, n_pad, n_w, n_tail):
    n, hdim = h.shape
    chunk = src2.shape[1]
    h_max = n_w // 2
    rows_per_tile = n_pad // _NS

    mesh = plsc.VectorSubcoreMesh(core_axis_name="c", subcore_axis_name="s",
                                  num_cores=_NC, num_subcores=_NS)

    @functools.partial(
        pl.kernel,
        out_type=jax.ShapeDtypeStruct((_NC, n_pad, hdim), jnp.float32),
        mesh=mesh,
        scratch_types=[
            pltpu.VMEM((h_max, chunk), jnp.int32),
            pltpu.VMEM((h_max, chunk), jnp.int32),
            pltpu.VMEM((chunk, hdim), jnp.float32),
            pltpu.VMEM((chunk, hdim), jnp.float32),
            pltpu.VMEM_SHARED((n_pad, hdim), jnp.float32),
            pltpu.SemaphoreType.DMA,
            pltpu.SemaphoreType.DMA,
        ],
    )
    def k(h_hbm, src_hbm, dst_hbm, out_hbm, src_v, dst_v, rows_v, rows_b,
          agg_sh, sem_a, sem_b):
        cid = lax.axis_index("c")
        sid = lax.axis_index("s")

        with jax.named_scope("agg_zero"):
            @pl.loop(0, chunk)
            def _(r):
                @pl.loop(0, hdim, step=16)
                def _(c0):
                    rows_v[r, pl.ds(c0, 16)] = jnp.zeros((16,), jnp.float32)

            @pl.loop(0, rows_per_tile, step=chunk)
            def _(j):
                pltpu.sync_copy(rows_v,
                                agg_sh.at[pl.ds(sid * rows_per_tile + j, chunk)])

            plsc.subcore_barrier()

        def gather(c, buf, sem):
            pltpu.async_copy(h_hbm.at[src_v.at[c]], buf, sem)

        def gather_wait(c, buf, sem):
            pltpu.make_async_copy(h_hbm.at[src_v.at[c]], buf, sem).wait()

        def scatter(c, buf):
            pltpu.sync_copy(buf, agg_sh.at[dst_v.at[c]], add=True)

        def run_core(nc, off):
            n_stage = -(-nc // h_max)
            for s in range(n_stage):
                hc = min(h_max, nc - s * h_max)
                base = off + s * h_max
                pltpu.sync_copy(src_hbm.at[pl.ds(base, hc)],
                                src_v.at[pl.ds(0, hc)])
                pltpu.sync_copy(dst_hbm.at[pl.ds(base, hc)],
                                dst_v.at[pl.ds(0, hc)])

                gather(0, rows_v, sem_a)

                @pl.loop(0, hc - 2, step=2)
                def _(c):
                    gather(c + 1, rows_b, sem_b)
                    gather_wait(c, rows_v, sem_a)
                    scatter(c, rows_v)
                    gather(c + 2, rows_v, sem_a)
                    gather_wait(c + 1, rows_b, sem_b)
                    scatter(c + 1, rows_b)

                gather(hc - 1, rows_b, sem_b)
                gather_wait(hc - 2, rows_v, sem_a)
                scatter(hc - 2, rows_v)
                gather_wait(hc - 1, rows_b, sem_b)
                scatter(hc - 1, rows_b)

        w = cid * _NS + sid
        with jax.named_scope("edge_loop"):
            if n_tail == n_w:
                run_core(n_w, w * n_w)
            else:
                @pl.when(w != _NW - 1)
                def _():
                    run_core(n_w, w * n_w)

                @pl.when(w == _NW - 1)
                def _():
                    run_core(n_tail, (_NW - 1) * n_w)

            plsc.subcore_barrier()

        with jax.named_scope("agg_out"):
            @pl.loop(0, rows_per_tile, step=chunk)
            def _(j):
                r = sid * rows_per_tile + j
                pltpu.sync_copy(agg_sh.at[pl.ds(r, chunk)],
                                out_hbm.at[cid, pl.ds(r, chunk)])

    return k(h, src2, dst2)


def _mlp_body(agg_ref, h_ref, coef_ref, w1_ref, b1_ref, w2_ref, b2_ref,
              sc_ref, sh_ref, o_ref):
    a = agg_ref[0] + agg_ref[1] + coef_ref[...] * h_ref[...]
    z = jnp.dot(a, w1_ref[...], preferred_element_type=jnp.float32)
    z = jnp.maximum(z + b1_ref[...], 0.0)
    z = jnp.dot(z, w2_ref[...], preferred_element_type=jnp.float32)
    z = jnp.maximum(z + b2_ref[...], 0.0)
    o_ref[...] = z * sc_ref[...] + sh_ref[...]


def _mlp(aggp, h, coef_row, w1, b1, w2, b2, scale, shift, blk=2000):
    n, hd = h.shape
    grid = (n // blk,)
    full = lambda i: (0, 0)
    return pl.pallas_call(
        _mlp_body,
        grid=grid,
        in_specs=[
            pl.BlockSpec((_NC, blk, hd), lambda i: (0, i, 0)),
            pl.BlockSpec((blk, hd), lambda i: (i, 0)),
            pl.BlockSpec((1, hd), full),
            pl.BlockSpec((hd, hd), full),
            pl.BlockSpec((1, hd), full),
            pl.BlockSpec((hd, hd), full),
            pl.BlockSpec((1, hd), full),
            pl.BlockSpec((1, hd), full),
            pl.BlockSpec((1, hd), full),
        ],
        out_specs=pl.BlockSpec((blk, hd), lambda i: (i, 0)),
        out_shape=jax.ShapeDtypeStruct((n, hd), jnp.float32),
    )(aggp, h, coef_row, w1, b1, w2, b2, scale, shift)


def _mlp_pool_body(agg_ref, h_ref, coef_ref, w1_ref, b1_ref, w2_ref, b2_ref,
                   sc_ref, sh_ref, batch_ref, lw1_ref, lb1_ref, lw2_ref,
                   lb2_ref, o_ref, out_ref, ge_ref, sums_ref, cnt_ref):
    i = pl.program_id(0)
    a = agg_ref[0] + agg_ref[1] + coef_ref[...] * h_ref[...]
    z = jnp.dot(a, w1_ref[...], preferred_element_type=jnp.float32)
    z = jnp.maximum(z + b1_ref[...], 0.0)
    z = jnp.dot(z, w2_ref[...], preferred_element_type=jnp.float32)
    z = jnp.maximum(z + b2_ref[...], 0.0)
    hb = z * sc_ref[...] + sh_ref[...]
    o_ref[...] = hb

    seg = lax.broadcasted_iota(jnp.int32, (1, _G), 1)
    onehot = (batch_ref[...] == seg).astype(jnp.float32)
    part = lax.dot_general(onehot, hb, (((0,), (0,)), ((), ())),
                           preferred_element_type=jnp.float32)
    pc = jnp.sum(onehot, axis=0)[:, None]

    @pl.when(i == 0)
    def _():
        sums_ref[...] = part
        cnt_ref[...] = pc

    @pl.when(i > 0)
    def _():
        sums_ref[...] += part
        cnt_ref[...] += pc

    @pl.when(i == pl.num_programs(0) - 1)
    def _():
        ge = sums_ref[...] / jnp.maximum(cnt_ref[...], 1.0)
        zz = jnp.dot(ge, lw1_ref[...], preferred_element_type=jnp.float32)
        zz = jnp.maximum(zz + lb1_ref[...], 0.0)
        out = jnp.dot(zz, lw2_ref[...],
                      preferred_element_type=jnp.float32) + lb2_ref[...]
        ge_ref[...] = ge
        out_ref[...] = out


def _mlp_pool(aggp, h, coef_row, w1, b1, w2, b2, scale, shift,
              batch2, lw1, lb1, lw2, lb2, blk=2000):
    n, hd = h.shape
    c = lw2.shape[1]
    grid = (n // blk,)
    full = lambda i: (0, 0)
    return pl.pallas_call(
        _mlp_pool_body,
        grid=grid,
        in_specs=[
            pl.BlockSpec((_NC, blk, hd), lambda i: (0, i, 0)),
            pl.BlockSpec((blk, hd), lambda i: (i, 0)),
            pl.BlockSpec((1, hd), full),
            pl.BlockSpec((hd, hd), full),
            pl.BlockSpec((1, hd), full),
            pl.BlockSpec((hd, hd), full),
            pl.BlockSpec((1, hd), full),
            pl.BlockSpec((1, hd), full),
            pl.BlockSpec((1, hd), full),
            pl.BlockSpec((blk, 1), lambda i: (i, 0)),
            pl.BlockSpec((hd, hd), full),
            pl.BlockSpec((1, hd), full),
            pl.BlockSpec((hd, c), full),
            pl.BlockSpec((1, c), full),
        ],
        out_specs=(
            pl.BlockSpec((blk, hd), lambda i: (i, 0)),
            pl.BlockSpec((_G, c), full),
            pl.BlockSpec((_G, hd), full),
        ),
        out_shape=(
            jax.ShapeDtypeStruct((n, hd), jnp.float32),
            jax.ShapeDtypeStruct((_G, c), jnp.float32),
            jax.ShapeDtypeStruct((_G, hd), jnp.float32),
        ),
        scratch_shapes=[
            pltpu.VMEM((_G, hd), jnp.float32),
            pltpu.VMEM((_G, 1), jnp.float32),
        ],
    )(aggp, h, coef_row, w1, b1, w2, b2, scale, shift,
      batch2, lw1, lb1, lw2, lb2)


def kernel(x, edge_index, batch, params):
    n, _ = x.shape
    e = edge_index.shape[1]
    hd = params["l0"]["W1"].shape[1]

    n_pad = -(-n // (_NS * _CHUNK)) * (_NS * _CHUNK)
    nf, rem = divmod(e, _CHUNK)
    n_w = (-(-nf // _NW) + 7) // 8 * 8
    n_tail = nf - (_NW - 1) * n_w
    if rem == 0 and 0 < n_tail <= n_w and n_tail % 4 == 0 and n_w % 4 == 0:
        src2 = edge_index[0].astype(jnp.int32).reshape(nf, _CHUNK)
        dst2 = edge_index[1].astype(jnp.int32).reshape(nf, _CHUNK)
    else:
        e_pad = _NW * n_w * _CHUNK
        pad = e_pad - e
        pad_ids = jnp.arange(pad, dtype=jnp.int32)
        src2 = jnp.concatenate(
            [edge_index[0].astype(jnp.int32), pad_ids % n]).reshape(-1, _CHUNK)
        dst2 = jnp.concatenate(
            [edge_index[1].astype(jnp.int32),
             n + pad_ids % (n_pad - n)]).reshape(-1, _CHUNK)
        n_tail = n_w

    h = x
    for l in range(3):
        p = params["l%d" % l]
        aggp = _sc_aggregate(h, src2, dst2, n_pad, n_w, n_tail)
        coef = jnp.broadcast_to((1.0 + p["eps"]).reshape(1, 1), (1, hd))
        scale = (p["gamma"] / jnp.sqrt(p["var"] + _BN_EPS)).reshape(1, hd)
        shift = p["beta"].reshape(1, hd) - p["mean"].reshape(1, hd) * scale
        if l < 2:
            h = _mlp(aggp, h, coef,
                     p["W1"], p["b1"].reshape(1, hd),
                     p["W2"], p["b2"].reshape(1, hd),
                     scale, shift)
        else:
            h, out, ge = _mlp_pool(
                aggp, h, coef,
                p["W1"], p["b1"].reshape(1, hd),
                p["W2"], p["b2"].reshape(1, hd),
                scale, shift,
                batch.astype(jnp.int32).reshape(n, 1),
                params["lin1_W"], params["lin1_b"].reshape(1, hd),
                params["lin2_W"], params["lin2_b"].reshape(1, -1))
    return (out, h, ge)
